# baseline (speedup 1.0000x reference)
import functools
import math

import jax
import jax.numpy as jnp
from jax.experimental import pallas as pl
from jax.experimental.pallas import tpu as pltpu

_VMEM_LIMIT = 56 * 1024 * 1024
_EPS = 1e-5


def _conv_body(*refs, n_a, n_w, taps, M, relu, gn, temb, n_par, cg):
    idx = 0
    a_refs = refs[idx:idx + n_a]; idx += n_a
    w_refs = refs[idx:idx + n_w]; idx += n_w
    b_ref = refs[idx]; idx += 1
    if temb:
        comb_ref, tw_ref, tb_ref = refs[idx:idx + 3]; idx += 3
    if gn:
        g_ref, be_ref = refs[idx:idx + 2]; idx += 2
    o_ref = refs[idx]
    tn = o_ref.shape[-1]

    accs = [None] * n_par
    for (ai, ro, co, cl, wi, ws, par) in taps:
        a = a_refs[ai][0, ro:ro + M, co:co + cl]
        w = w_refs[wi][ws]
        d = jnp.dot(a, w, preferred_element_type=jnp.float32)
        accs[par] = d if accs[par] is None else accs[par] + d

    bias = b_ref[...]
    for par in range(n_par):
        z = accs[par] + bias
        if relu:
            z = jnp.maximum(z, 0.0)
        if gn:
            zb = z.astype(jnp.bfloat16).astype(jnp.float32)
            s1 = jnp.sum(zb, axis=0, keepdims=True)
            s2 = jnp.sum(zb * zb, axis=0, keepdims=True)
            grp = jax.lax.broadcasted_iota(jnp.int32, (1, tn), 1) // cg
            inv_n = 1.0 / float(M * cg)
            mean = jnp.zeros((1, tn), jnp.float32)
            ex2 = jnp.zeros((1, tn), jnp.float32)
            for g in range(tn // cg):
                m = grp == g
                gs1 = jnp.sum(jnp.where(m, s1, 0.0), axis=-1, keepdims=True)
                gs2 = jnp.sum(jnp.where(m, s2, 0.0), axis=-1, keepdims=True)
                mean = jnp.where(m, gs1 * inv_n, mean)
                ex2 = jnp.where(m, gs2 * inv_n, ex2)
            var = jnp.maximum(ex2 - mean * mean, 0.0)
            inv_std = jax.lax.rsqrt(var + _EPS)
            z = (zb - mean) * inv_std * g_ref[...] + be_ref[...]
            if temb:
                t = jnp.dot(comb_ref[0], tw_ref[...],
                            preferred_element_type=jnp.float32) + tb_ref[...]
                t = jnp.maximum(t, 0.0).astype(jnp.bfloat16).astype(jnp.float32)
                z = z + t
        o_ref[0, par] = z.astype(o_ref.dtype)


def _conv_call(a_list, w_list, bias, *, taps, M, Cout, relu=False, gn=None,
               temb=None, n_par=1, J=1, out_dtype=jnp.bfloat16):
    N = a_list[0].shape[0]
    tn = Cout // J
    bl = J == 1
    ix = (lambda f: (lambda i, j: f(j, i))) if bl else (lambda f: f)
    grid = (N, J) if bl else (J, N)
    in_specs = []
    args = []
    for a in a_list:
        R, K = a.shape[1], a.shape[2]
        in_specs.append(pl.BlockSpec((1, R, K), ix(lambda j, i: (i, 0, 0))))
        args.append(a)
    for w in w_list:
        T, K = w.shape[0], w.shape[1]
        in_specs.append(pl.BlockSpec((T, K, tn), ix(lambda j, i: (0, 0, j))))
        args.append(w)
    in_specs.append(pl.BlockSpec((1, tn), ix(lambda j, i: (0, j))))
    args.append(bias.astype(jnp.float32).reshape(1, Cout))
    if temb is not None:
        comb, tw, tb = temb
        in_specs += [pl.BlockSpec((1, 1, 32), ix(lambda j, i: (i, 0, 0))),
                     pl.BlockSpec((32, tn), ix(lambda j, i: (0, j))),
                     pl.BlockSpec((1, tn), ix(lambda j, i: (0, j)))]
        args += [comb.reshape(N, 1, 32), tw,
                 tb.astype(jnp.float32).reshape(1, Cout)]
    cg = 0
    if gn is not None:
        g, b = gn
        cg = Cout // 8
        in_specs += [pl.BlockSpec((1, tn), ix(lambda j, i: (0, j))),
                     pl.BlockSpec((1, tn), ix(lambda j, i: (0, j)))]
        args += [g.astype(jnp.float32).reshape(1, Cout),
                 b.astype(jnp.float32).reshape(1, Cout)]
    return pl.pallas_call(
        functools.partial(_conv_body, n_a=len(a_list), n_w=len(w_list),
                          taps=taps, M=M, relu=relu, gn=gn is not None,
                          temb=temb is not None, n_par=n_par, cg=cg),
        out_shape=jax.ShapeDtypeStruct((N, n_par, M, Cout), out_dtype),
        grid=grid,
        in_specs=in_specs,
        out_specs=pl.BlockSpec((1, n_par, M, tn), ix(lambda j, i: (i, 0, 0, j))),
        compiler_params=pltpu.CompilerParams(
            dimension_semantics=("parallel", "parallel"),
            vmem_limit_bytes=_VMEM_LIMIT),
    )(*args)


def _gn_epilogue(z, g, be, tn, cg):
    zb = z.astype(jnp.bfloat16).astype(jnp.float32)
    M = zb.shape[0]
    s1 = jnp.sum(zb, axis=0, keepdims=True)
    s2 = jnp.sum(zb * zb, axis=0, keepdims=True)
    grp = jax.lax.broadcasted_iota(jnp.int32, (1, tn), 1) // cg
    inv_n = 1.0 / float(M * cg)
    mean = jnp.zeros((1, tn), jnp.float32)
    ex2 = jnp.zeros((1, tn), jnp.float32)
    for g_i in range(tn // cg):
        m = grp == g_i
        gs1 = jnp.sum(jnp.where(m, s1, 0.0), axis=-1, keepdims=True)
        gs2 = jnp.sum(jnp.where(m, s2, 0.0), axis=-1, keepdims=True)
        mean = jnp.where(m, gs1 * inv_n, mean)
        ex2 = jnp.where(m, gs2 * inv_n, ex2)
    var = jnp.maximum(ex2 - mean * mean, 0.0)
    inv_std = jax.lax.rsqrt(var + _EPS)
    return (zb - mean) * inv_std * g + be


def _dbl_body(a_ref, w1_ref, w2_ref, b1_ref, comb_ref, tw_ref, tb_ref,
              g1_ref, be1_ref, b2_ref, g2_ref, be2_ref, o_ref, *, M, W, C1,
              cg2):
    acc = None
    for dh in range(3):
        d = jnp.dot(a_ref[0, dh * W:dh * W + M, :], w1_ref[dh],
                    preferred_element_type=jnp.float32)
        acc = d if acc is None else acc + d
    z = jnp.maximum(acc + b1_ref[...], 0.0)
    h = _gn_epilogue(z, g1_ref[...], be1_ref[...], C1, C1 // 8)
    t = jnp.dot(comb_ref[0], tw_ref[...],
                preferred_element_type=jnp.float32) + tb_ref[...]
    t = jnp.maximum(t, 0.0).astype(jnp.bfloat16).astype(jnp.float32)
    h1b = (h + t).astype(jnp.bfloat16)

    zc = jnp.zeros((1, C1), jnp.bfloat16)
    h1z = jnp.concatenate([zc, h1b, zc], axis=0)
    jm = jax.lax.broadcasted_iota(jnp.int32, (M, 1), 0) % W
    s0 = jnp.where(jm != 0, h1z[0:M], jnp.zeros_like(h1b))
    s2 = jnp.where(jm != W - 1, h1z[2:M + 2], jnp.zeros_like(h1b))
    h3 = jnp.concatenate([s0, h1b, s2], axis=1)
    zr = jnp.zeros((W, 3 * C1), jnp.bfloat16)
    h3z = jnp.concatenate([zr, h3, zr], axis=0)

    acc2 = None
    for dh in range(3):
        d = jnp.dot(h3z[dh * W:dh * W + M, :], w2_ref[dh],
                    preferred_element_type=jnp.float32)
        acc2 = d if acc2 is None else acc2 + d
    tn = o_ref.shape[-1]
    z2 = jnp.maximum(acc2 + b2_ref[...], 0.0)
    out = _gn_epilogue(z2, g2_ref[...], be2_ref[...], tn, cg2)
    o_ref[0, 0] = out.astype(o_ref.dtype)


def _pad_shift3(x):
    N, H, W, C = x.shape
    xp = jnp.pad(x, ((0, 0), (1, 1), (1, 1), (0, 0)))
    xs = jnp.concatenate([xp[:, :, d:d + W, :] for d in range(3)], axis=-1)
    return xs.reshape(N, (H + 2) * W, 3 * C)


def _conv3(x, w, b, *, relu, gn=None, temb=None, J=1):
    N, H, W, C = x.shape
    Cout = w.shape[-1]
    A = _pad_shift3(x)
    w3 = w.reshape(3, 3 * C, Cout)
    taps = [(0, d * W, 0, 3 * C, 0, d, 0) for d in range(3)]
    out = _conv_call([A], [w3], b, taps=taps, M=H * W, Cout=Cout, relu=relu,
                     gn=gn, temb=temb, J=J)
    return out.reshape(N, H, W, Cout)


def _conv3_pair(x, p, comb, *, J=1):
    N, H, W, Cin = x.shape
    C1 = p['conv1_w'].shape[-1]
    C2 = p['conv2_w'].shape[-1]
    M = H * W
    tn = C2 // J
    bl = J == 1
    ix = (lambda f: (lambda i, j: f(j, i))) if bl else (lambda f: f)
    grid = (N, J) if bl else (J, N)
    A = _pad_shift3(x)
    R, K1 = A.shape[1], A.shape[2]
    w1 = p['conv1_w'].reshape(3, K1, C1)
    w2 = p['conv2_w'].reshape(3, 3 * C1, C2)
    f32 = lambda a: a.astype(jnp.float32)
    args = [A, w1, w2,
            f32(p['conv1_b']).reshape(1, C1),
            comb.reshape(N, 1, 32), p['time_w'],
            f32(p['time_b']).reshape(1, C1),
            f32(p['gn1_g']).reshape(1, C1), f32(p['gn1_b']).reshape(1, C1),
            f32(p['conv2_b']).reshape(1, C2),
            f32(p['gn2_g']).reshape(1, C2), f32(p['gn2_b']).reshape(1, C2)]
    in_specs = [
        pl.BlockSpec((1, R, K1), ix(lambda j, i: (i, 0, 0))),
        pl.BlockSpec((3, K1, C1), ix(lambda j, i: (0, 0, 0))),
        pl.BlockSpec((3, 3 * C1, tn), ix(lambda j, i: (0, 0, j))),
        pl.BlockSpec((1, C1), ix(lambda j, i: (0, 0))),
        pl.BlockSpec((1, 1, 32), ix(lambda j, i: (i, 0, 0))),
        pl.BlockSpec((32, C1), ix(lambda j, i: (0, 0))),
        pl.BlockSpec((1, C1), ix(lambda j, i: (0, 0))),
        pl.BlockSpec((1, C1), ix(lambda j, i: (0, 0))),
        pl.BlockSpec((1, C1), ix(lambda j, i: (0, 0))),
        pl.BlockSpec((1, tn), ix(lambda j, i: (0, j))),
        pl.BlockSpec((1, tn), ix(lambda j, i: (0, j))),
        pl.BlockSpec((1, tn), ix(lambda j, i: (0, j))),
    ]
    out = pl.pallas_call(
        functools.partial(_dbl_body, M=M, W=W, C1=C1, cg2=C2 // 8),
        out_shape=jax.ShapeDtypeStruct((N, 1, M, C2), jnp.bfloat16),
        grid=grid,
        in_specs=in_specs,
        out_specs=pl.BlockSpec((1, 1, M, tn), ix(lambda j, i: (i, 0, 0, j))),
        compiler_params=pltpu.CompilerParams(
            dimension_semantics=("parallel", "parallel"),
            vmem_limit_bytes=_VMEM_LIMIT),
    )(*args)
    return out.reshape(N, H, W, C2)


def _down4(x, w, b, *, J=1):
    N, H, W, C = x.shape
    Ho, Wo = H // 2, W // 2
    Cout = w.shape[-1]
    xp = jnp.pad(x, ((0, 0), (1, 2), (1, 2), (0, 0)))
    xs = jnp.concatenate([xp[:, :, d::2, :][:, :, :Wo, :] for d in range(4)],
                         axis=-1)
    A0 = xs[:, 0::2].reshape(N, -1, 4 * C)
    A1 = xs[:, 1::2].reshape(N, -1, 4 * C)
    w4 = w.reshape(4, 4 * C, Cout)
    taps = [(d % 2, (d // 2) * Wo, 0, 4 * C, 0, d, 0) for d in range(4)]
    out = _conv_call([A0, A1], [w4], b, taps=taps, M=Ho * Wo, Cout=Cout, J=J)
    return out.reshape(N, Ho, Wo, Cout)


def _transconv(x, wlist4, b, *, J=1):
    N, H, W, C = x.shape
    Cout = wlist4[0].shape[-1]
    A = _pad_shift3(x)
    ws = [w.reshape(2, 2 * C, Cout) for w in wlist4]
    taps = []
    for r in range(2):
        for c in range(2):
            p = r * 2 + c
            for a in range(2):
                taps.append((0, (r + a) * W, c * C, 2 * C, p, a, p))
    out = _conv_call([A], ws, b, taps=taps, M=H * W, Cout=Cout, n_par=4, J=J)
    out = out.reshape(N, 2, 2, H, W, Cout)
    out = out.transpose(0, 3, 1, 4, 2, 5).reshape(N, 2 * H, 2 * W, Cout)
    return out


def _conv1x1(x, w, b, out_dtype):
    N, H, W, C = x.shape
    Cout = w.shape[-1]
    A = x.reshape(N, H * W, C)
    w1 = w.reshape(1, C, Cout)
    out = _conv_call([A], [w1], b, taps=[(0, 0, 0, C, 0, 0, 0)], M=H * W,
                     Cout=Cout, out_dtype=out_dtype)
    return out.reshape(N, H, W, Cout)


def _block(x, p, comb, *, up, J1=1, J2=1, Jt=1):
    h = _conv3_pair(x, p, comb, J=J2)
    if up:
        return _transconv(h, p['trans_w'], p['trans_b'], J=Jt)
    return _down4(h, p['trans_w'], p['trans_b'], J=Jt)


def kernel(time_mlp_w, time_mlp_b, label_emb, conv0_w, conv0_b, out_w, out_b, down0_conv1_w, down0_conv1_b, down0_conv2_w, down0_conv2_b, down0_time_w, down0_time_b, down0_trans_w, down0_trans_b, down0_gn1_g, down0_gn1_b, down0_gn2_g, down0_gn2_b, down1_conv1_w, down1_conv1_b, down1_conv2_w, down1_conv2_b, down1_time_w, down1_time_b, down1_trans_w, down1_trans_b, down1_gn1_g, down1_gn1_b, down1_gn2_g, down1_gn2_b, down2_conv1_w, down2_conv1_b, down2_conv2_w, down2_conv2_b, down2_time_w, down2_time_b, down2_trans_w, down2_trans_b, down2_gn1_g, down2_gn1_b, down2_gn2_g, down2_gn2_b, down3_conv1_w, down3_conv1_b, down3_conv2_w, down3_conv2_b, down3_time_w, down3_time_b, down3_trans_w, down3_trans_b, down3_gn1_g, down3_gn1_b, down3_gn2_g, down3_gn2_b, up0_conv1_w, up0_conv1_b, up0_conv2_w, up0_conv2_b, up0_time_w, up0_time_b, up0_trans_b, up0_gn1_g, up0_gn1_b, up0_gn2_g, up0_gn2_b, up0_trans_w_0_0, up0_trans_w_0_1, up0_trans_w_1_0, up0_trans_w_1_1, up1_conv1_w, up1_conv1_b, up1_conv2_w, up1_conv2_b, up1_time_w, up1_time_b, up1_trans_b, up1_gn1_g, up1_gn1_b, up1_gn2_g, up1_gn2_b, up1_trans_w_0_0, up1_trans_w_0_1, up1_trans_w_1_0, up1_trans_w_1_1, up2_conv1_w, up2_conv1_b, up2_conv2_w, up2_conv2_b, up2_time_w, up2_time_b, up2_trans_b, up2_gn1_g, up2_gn1_b, up2_gn2_g, up2_gn2_b, up2_trans_w_0_0, up2_trans_w_0_1, up2_trans_w_1_0, up2_trans_w_1_1, up3_conv1_w, up3_conv1_b, up3_conv2_w, up3_conv2_b, up3_time_w, up3_time_b, up3_trans_b, up3_gn1_g, up3_gn1_b, up3_gn2_g, up3_gn2_b, up3_trans_w_0_0, up3_trans_w_0_1, up3_trans_w_1_0, up3_trans_w_1_1, x, timestep, y):
    downs = [
        dict(conv1_w=down0_conv1_w, conv1_b=down0_conv1_b, conv2_w=down0_conv2_w,
             conv2_b=down0_conv2_b, time_w=down0_time_w, time_b=down0_time_b,
             trans_w=down0_trans_w, trans_b=down0_trans_b, gn1_g=down0_gn1_g,
             gn1_b=down0_gn1_b, gn2_g=down0_gn2_g, gn2_b=down0_gn2_b),
        dict(conv1_w=down1_conv1_w, conv1_b=down1_conv1_b, conv2_w=down1_conv2_w,
             conv2_b=down1_conv2_b, time_w=down1_time_w, time_b=down1_time_b,
             trans_w=down1_trans_w, trans_b=down1_trans_b, gn1_g=down1_gn1_g,
             gn1_b=down1_gn1_b, gn2_g=down1_gn2_g, gn2_b=down1_gn2_b),
        dict(conv1_w=down2_conv1_w, conv1_b=down2_conv1_b, conv2_w=down2_conv2_w,
             conv2_b=down2_conv2_b, time_w=down2_time_w, time_b=down2_time_b,
             trans_w=down2_trans_w, trans_b=down2_trans_b, gn1_g=down2_gn1_g,
             gn1_b=down2_gn1_b, gn2_g=down2_gn2_g, gn2_b=down2_gn2_b),
        dict(conv1_w=down3_conv1_w, conv1_b=down3_conv1_b, conv2_w=down3_conv2_w,
             conv2_b=down3_conv2_b, time_w=down3_time_w, time_b=down3_time_b,
             trans_w=down3_trans_w, trans_b=down3_trans_b, gn1_g=down3_gn1_g,
             gn1_b=down3_gn1_b, gn2_g=down3_gn2_g, gn2_b=down3_gn2_b),
    ]
    ups = [
        dict(conv1_w=up0_conv1_w, conv1_b=up0_conv1_b, conv2_w=up0_conv2_w,
             conv2_b=up0_conv2_b, time_w=up0_time_w, time_b=up0_time_b,
             trans_w=[up0_trans_w_0_0, up0_trans_w_0_1, up0_trans_w_1_0,
                      up0_trans_w_1_1], trans_b=up0_trans_b, gn1_g=up0_gn1_g,
             gn1_b=up0_gn1_b, gn2_g=up0_gn2_g, gn2_b=up0_gn2_b),
        dict(conv1_w=up1_conv1_w, conv1_b=up1_conv1_b, conv2_w=up1_conv2_w,
             conv2_b=up1_conv2_b, time_w=up1_time_w, time_b=up1_time_b,
             trans_w=[up1_trans_w_0_0, up1_trans_w_0_1, up1_trans_w_1_0,
                      up1_trans_w_1_1], trans_b=up1_trans_b, gn1_g=up1_gn1_g,
             gn1_b=up1_gn1_b, gn2_g=up1_gn2_g, gn2_b=up1_gn2_b),
        dict(conv1_w=up2_conv1_w, conv1_b=up2_conv1_b, conv2_w=up2_conv2_w,
             conv2_b=up2_conv2_b, time_w=up2_time_w, time_b=up2_time_b,
             trans_w=[up2_trans_w_0_0, up2_trans_w_0_1, up2_trans_w_1_0,
                      up2_trans_w_1_1], trans_b=up2_trans_b, gn1_g=up2_gn1_g,
             gn1_b=up2_gn1_b, gn2_g=up2_gn2_g, gn2_b=up2_gn2_b),
        dict(conv1_w=up3_conv1_w, conv1_b=up3_conv1_b, conv2_w=up3_conv2_w,
             conv2_b=up3_conv2_b, time_w=up3_time_w, time_b=up3_time_b,
             trans_w=[up3_trans_w_0_0, up3_trans_w_0_1, up3_trans_w_1_0,
                      up3_trans_w_1_1], trans_b=up3_trans_b, gn1_g=up3_gn1_g,
             gn1_b=up3_gn1_b, gn2_g=up3_gn2_g, gn2_b=up3_gn2_b),
    ]

    xh = jnp.transpose(x, (0, 2, 3, 1)).astype(jnp.bfloat16)

    half = 16
    freqs = jnp.exp(jnp.arange(half, dtype=jnp.float32)
                    * -(math.log(10000.0) / (half - 1)))
    targs = timestep.astype(jnp.float32)[:, None] * freqs[None, :]
    t_emb = jnp.concatenate([jnp.sin(targs), jnp.cos(targs)], axis=-1)
    te = jnp.dot(t_emb.astype(jnp.bfloat16), time_mlp_w,
                 preferred_element_type=jnp.float32) \
        + time_mlp_b.astype(jnp.float32)[None, :]
    te = jnp.maximum(te, 0.0)
    comb = (te + label_emb[y]).astype(jnp.bfloat16)

    h = _conv3(xh, conv0_w, conv0_b, relu=False)

    down_J = [(1, 1, 1), (1, 1, 1), (1, 1, 1), (1, 2, 2)]
    up_J = [(2, 1, 1), (1, 1, 1), (1, 1, 1), (1, 1, 1)]

    residuals = []
    for p, (j1, j2, jt) in zip(downs, down_J):
        h = _block(h, p, comb, up=False, J1=j1, J2=j2, Jt=jt)
        residuals.append(h)
    for p, (j1, j2, jt) in zip(ups, up_J):
        r = residuals.pop()
        h = _block(jnp.concatenate([h, r], axis=-1), p, comb, up=True,
                   J1=j1, J2=j2, Jt=jt)

    out = _conv1x1(h, out_w, out_b, jnp.float32)
    return jnp.transpose(out, (0, 3, 1, 2))


# speedup vs baseline: 1.0180x; 1.0180x over previous
import functools
import math

import jax
import jax.numpy as jnp
from jax.experimental import pallas as pl
from jax.experimental.pallas import tpu as pltpu

_VMEM_LIMIT = 56 * 1024 * 1024
_EPS = 1e-5


def _conv_body(*refs, n_a, n_w, taps, M, relu, gn, temb, n_par, cg):
    idx = 0
    a_refs = refs[idx:idx + n_a]; idx += n_a
    w_refs = refs[idx:idx + n_w]; idx += n_w
    b_ref = refs[idx]; idx += 1
    if temb:
        comb_ref, tw_ref, tb_ref = refs[idx:idx + 3]; idx += 3
    if gn:
        g_ref, be_ref = refs[idx:idx + 2]; idx += 2
    o_ref = refs[idx]
    tn = o_ref.shape[-1]

    accs = [None] * n_par
    for (ai, ro, co, cl, wi, ws, par) in taps:
        a = a_refs[ai][0, ro:ro + M, co:co + cl]
        w = w_refs[wi][ws]
        d = jnp.dot(a, w, preferred_element_type=jnp.float32)
        accs[par] = d if accs[par] is None else accs[par] + d

    bias = b_ref[...]
    for par in range(n_par):
        z = accs[par] + bias
        if relu:
            z = jnp.maximum(z, 0.0)
        if gn:
            zb = z.astype(jnp.bfloat16).astype(jnp.float32)
            s1 = jnp.sum(zb, axis=0, keepdims=True)
            s2 = jnp.sum(zb * zb, axis=0, keepdims=True)
            grp = jax.lax.broadcasted_iota(jnp.int32, (1, tn), 1) // cg
            inv_n = 1.0 / float(M * cg)
            mean = jnp.zeros((1, tn), jnp.float32)
            ex2 = jnp.zeros((1, tn), jnp.float32)
            for g in range(tn // cg):
                m = grp == g
                gs1 = jnp.sum(jnp.where(m, s1, 0.0), axis=-1, keepdims=True)
                gs2 = jnp.sum(jnp.where(m, s2, 0.0), axis=-1, keepdims=True)
                mean = jnp.where(m, gs1 * inv_n, mean)
                ex2 = jnp.where(m, gs2 * inv_n, ex2)
            var = jnp.maximum(ex2 - mean * mean, 0.0)
            inv_std = jax.lax.rsqrt(var + _EPS)
            z = (zb - mean) * inv_std * g_ref[...] + be_ref[...]
            if temb:
                t = jnp.dot(comb_ref[0], tw_ref[...],
                            preferred_element_type=jnp.float32) + tb_ref[...]
                t = jnp.maximum(t, 0.0).astype(jnp.bfloat16).astype(jnp.float32)
                z = z + t
        o_ref[0, par] = z.astype(o_ref.dtype)


def _conv_call(a_list, w_list, bias, *, taps, M, Cout, relu=False, gn=None,
               temb=None, n_par=1, J=1, out_dtype=jnp.bfloat16):
    N = a_list[0].shape[0]
    tn = Cout // J
    bl = J == 1
    ix = (lambda f: (lambda i, j: f(j, i))) if bl else (lambda f: f)
    grid = (N, J) if bl else (J, N)
    in_specs = []
    args = []
    for a in a_list:
        R, K = a.shape[1], a.shape[2]
        in_specs.append(pl.BlockSpec((1, R, K), ix(lambda j, i: (i, 0, 0))))
        args.append(a)
    for w in w_list:
        T, K = w.shape[0], w.shape[1]
        in_specs.append(pl.BlockSpec((T, K, tn), ix(lambda j, i: (0, 0, j))))
        args.append(w)
    in_specs.append(pl.BlockSpec((1, tn), ix(lambda j, i: (0, j))))
    args.append(bias.astype(jnp.float32).reshape(1, Cout))
    if temb is not None:
        comb, tw, tb = temb
        in_specs += [pl.BlockSpec((1, 1, 32), ix(lambda j, i: (i, 0, 0))),
                     pl.BlockSpec((32, tn), ix(lambda j, i: (0, j))),
                     pl.BlockSpec((1, tn), ix(lambda j, i: (0, j)))]
        args += [comb.reshape(N, 1, 32), tw,
                 tb.astype(jnp.float32).reshape(1, Cout)]
    cg = 0
    if gn is not None:
        g, b = gn
        cg = Cout // 8
        in_specs += [pl.BlockSpec((1, tn), ix(lambda j, i: (0, j))),
                     pl.BlockSpec((1, tn), ix(lambda j, i: (0, j)))]
        args += [g.astype(jnp.float32).reshape(1, Cout),
                 b.astype(jnp.float32).reshape(1, Cout)]
    return pl.pallas_call(
        functools.partial(_conv_body, n_a=len(a_list), n_w=len(w_list),
                          taps=taps, M=M, relu=relu, gn=gn is not None,
                          temb=temb is not None, n_par=n_par, cg=cg),
        out_shape=jax.ShapeDtypeStruct((N, n_par, M, Cout), out_dtype),
        grid=grid,
        in_specs=in_specs,
        out_specs=pl.BlockSpec((1, n_par, M, tn), ix(lambda j, i: (i, 0, 0, j))),
        compiler_params=pltpu.CompilerParams(
            dimension_semantics=("parallel", "parallel"),
            vmem_limit_bytes=_VMEM_LIMIT),
    )(*args)


def _gn_epilogue(z, g, be, tn, cg):
    zb = z.astype(jnp.bfloat16).astype(jnp.float32)
    M = zb.shape[0]
    s1 = jnp.sum(zb, axis=0, keepdims=True)
    s2 = jnp.sum(zb * zb, axis=0, keepdims=True)
    grp = jax.lax.broadcasted_iota(jnp.int32, (1, tn), 1) // cg
    inv_n = 1.0 / float(M * cg)
    mean = jnp.zeros((1, tn), jnp.float32)
    ex2 = jnp.zeros((1, tn), jnp.float32)
    for g_i in range(tn // cg):
        m = grp == g_i
        gs1 = jnp.sum(jnp.where(m, s1, 0.0), axis=-1, keepdims=True)
        gs2 = jnp.sum(jnp.where(m, s2, 0.0), axis=-1, keepdims=True)
        mean = jnp.where(m, gs1 * inv_n, mean)
        ex2 = jnp.where(m, gs2 * inv_n, ex2)
    var = jnp.maximum(ex2 - mean * mean, 0.0)
    inv_std = jax.lax.rsqrt(var + _EPS)
    return (zb - mean) * inv_std * g + be


def _dbl_body(a_ref, w1_ref, w2_ref, b1_ref, comb_ref, tw_ref, tb_ref,
              g1_ref, be1_ref, b2_ref, g2_ref, be2_ref, o_ref, *, M, W, C1,
              cg2, pack1, pack2):
    if pack1:
        P = jnp.dot(a_ref[0], w1_ref[...], preferred_element_type=jnp.float32)
        acc = (P[0:M, 0:C1] + P[W:W + M, C1:2 * C1]
               + P[2 * W:2 * W + M, 2 * C1:3 * C1])
    else:
        acc = None
        for dh in range(3):
            d = jnp.dot(a_ref[0, dh * W:dh * W + M, :], w1_ref[dh],
                        preferred_element_type=jnp.float32)
            acc = d if acc is None else acc + d
    z = jnp.maximum(acc + b1_ref[...], 0.0)
    h = _gn_epilogue(z, g1_ref[...], be1_ref[...], C1, C1 // 8)
    t = jnp.dot(comb_ref[0], tw_ref[...],
                preferred_element_type=jnp.float32) + tb_ref[...]
    t = jnp.maximum(t, 0.0).astype(jnp.bfloat16).astype(jnp.float32)
    h1b = (h + t).astype(jnp.bfloat16)

    zc = jnp.zeros((1, C1), jnp.bfloat16)
    h1z = jnp.concatenate([zc, h1b, zc], axis=0)
    jm = jax.lax.broadcasted_iota(jnp.int32, (M, 1), 0) % W
    s0 = jnp.where(jm != 0, h1z[0:M], jnp.zeros_like(h1b))
    s2 = jnp.where(jm != W - 1, h1z[2:M + 2], jnp.zeros_like(h1b))
    h3 = jnp.concatenate([s0, h1b, s2], axis=1)
    zr = jnp.zeros((W, 3 * C1), jnp.bfloat16)
    h3z = jnp.concatenate([zr, h3, zr], axis=0)

    tn = o_ref.shape[-1]
    if pack2:
        P2 = jnp.dot(h3z, w2_ref[...], preferred_element_type=jnp.float32)
        acc2 = (P2[0:M, 0:tn] + P2[W:W + M, tn:2 * tn]
                + P2[2 * W:2 * W + M, 2 * tn:3 * tn])
    else:
        acc2 = None
        for dh in range(3):
            d = jnp.dot(h3z[dh * W:dh * W + M, :], w2_ref[dh],
                        preferred_element_type=jnp.float32)
            acc2 = d if acc2 is None else acc2 + d
    z2 = jnp.maximum(acc2 + b2_ref[...], 0.0)
    out = _gn_epilogue(z2, g2_ref[...], be2_ref[...], tn, cg2)
    o_ref[0, 0] = out.astype(o_ref.dtype)


def _pad_shift3(x):
    N, H, W, C = x.shape
    xp = jnp.pad(x, ((0, 0), (1, 1), (1, 1), (0, 0)))
    xs = jnp.concatenate([xp[:, :, d:d + W, :] for d in range(3)], axis=-1)
    return xs.reshape(N, (H + 2) * W, 3 * C)


def _conv3(x, w, b, *, relu, gn=None, temb=None, J=1):
    N, H, W, C = x.shape
    Cout = w.shape[-1]
    A = _pad_shift3(x)
    w3 = w.reshape(3, 3 * C, Cout)
    taps = [(0, d * W, 0, 3 * C, 0, d, 0) for d in range(3)]
    out = _conv_call([A], [w3], b, taps=taps, M=H * W, Cout=Cout, relu=relu,
                     gn=gn, temb=temb, J=J)
    return out.reshape(N, H, W, Cout)


def _conv3_pair(x, p, comb, *, J=1):
    N, H, W, Cin = x.shape
    C1 = p['conv1_w'].shape[-1]
    C2 = p['conv2_w'].shape[-1]
    M = H * W
    tn = C2 // J
    bl = J == 1
    ix = (lambda f: (lambda i, j: f(j, i))) if bl else (lambda f: f)
    grid = (N, J) if bl else (J, N)
    A = _pad_shift3(x)
    R, K1 = A.shape[1], A.shape[2]
    pack1 = C1 <= 128
    pack2 = C2 <= 128 and J == 1
    w1 = p['conv1_w'].reshape(3, K1, C1)
    w2 = p['conv2_w'].reshape(3, 3 * C1, C2)
    if pack1:
        w1 = jnp.transpose(w1, (1, 0, 2)).reshape(K1, 3 * C1)
        w1_spec = pl.BlockSpec((K1, 3 * C1), ix(lambda j, i: (0, 0)))
    else:
        w1_spec = pl.BlockSpec((3, K1, C1), ix(lambda j, i: (0, 0, 0)))
    if pack2:
        w2 = jnp.transpose(w2, (1, 0, 2)).reshape(3 * C1, 3 * C2)
        w2_spec = pl.BlockSpec((3 * C1, 3 * C2), ix(lambda j, i: (0, 0)))
    else:
        w2_spec = pl.BlockSpec((3, 3 * C1, tn), ix(lambda j, i: (0, 0, j)))
    f32 = lambda a: a.astype(jnp.float32)
    args = [A, w1, w2,
            f32(p['conv1_b']).reshape(1, C1),
            comb.reshape(N, 1, 32), p['time_w'],
            f32(p['time_b']).reshape(1, C1),
            f32(p['gn1_g']).reshape(1, C1), f32(p['gn1_b']).reshape(1, C1),
            f32(p['conv2_b']).reshape(1, C2),
            f32(p['gn2_g']).reshape(1, C2), f32(p['gn2_b']).reshape(1, C2)]
    in_specs = [
        pl.BlockSpec((1, R, K1), ix(lambda j, i: (i, 0, 0))),
        w1_spec,
        w2_spec,
        pl.BlockSpec((1, C1), ix(lambda j, i: (0, 0))),
        pl.BlockSpec((1, 1, 32), ix(lambda j, i: (i, 0, 0))),
        pl.BlockSpec((32, C1), ix(lambda j, i: (0, 0))),
        pl.BlockSpec((1, C1), ix(lambda j, i: (0, 0))),
        pl.BlockSpec((1, C1), ix(lambda j, i: (0, 0))),
        pl.BlockSpec((1, C1), ix(lambda j, i: (0, 0))),
        pl.BlockSpec((1, tn), ix(lambda j, i: (0, j))),
        pl.BlockSpec((1, tn), ix(lambda j, i: (0, j))),
        pl.BlockSpec((1, tn), ix(lambda j, i: (0, j))),
    ]
    out = pl.pallas_call(
        functools.partial(_dbl_body, M=M, W=W, C1=C1, cg2=C2 // 8,
                          pack1=pack1, pack2=pack2),
        out_shape=jax.ShapeDtypeStruct((N, 1, M, C2), jnp.bfloat16),
        grid=grid,
        in_specs=in_specs,
        out_specs=pl.BlockSpec((1, 1, M, tn), ix(lambda j, i: (i, 0, 0, j))),
        compiler_params=pltpu.CompilerParams(
            dimension_semantics=("parallel", "parallel"),
            vmem_limit_bytes=_VMEM_LIMIT),
    )(*args)
    return out.reshape(N, H, W, C2)


def _down4(x, w, b, *, J=1):
    N, H, W, C = x.shape
    Ho, Wo = H // 2, W // 2
    Cout = w.shape[-1]
    xp = jnp.pad(x, ((0, 0), (1, 2), (1, 2), (0, 0)))
    xs = jnp.concatenate([xp[:, :, d::2, :][:, :, :Wo, :] for d in range(4)],
                         axis=-1)
    A0 = xs[:, 0::2].reshape(N, -1, 4 * C)
    A1 = xs[:, 1::2].reshape(N, -1, 4 * C)
    w4 = w.reshape(4, 4 * C, Cout)
    taps = [(d % 2, (d // 2) * Wo, 0, 4 * C, 0, d, 0) for d in range(4)]
    out = _conv_call([A0, A1], [w4], b, taps=taps, M=Ho * Wo, Cout=Cout, J=J)
    return out.reshape(N, Ho, Wo, Cout)


def _transconv(x, wlist4, b, *, J=1):
    N, H, W, C = x.shape
    Cout = wlist4[0].shape[-1]
    A = _pad_shift3(x)
    ws = [w.reshape(2, 2 * C, Cout) for w in wlist4]
    taps = []
    for r in range(2):
        for c in range(2):
            p = r * 2 + c
            for a in range(2):
                taps.append((0, (r + a) * W, c * C, 2 * C, p, a, p))
    out = _conv_call([A], ws, b, taps=taps, M=H * W, Cout=Cout, n_par=4, J=J)
    out = out.reshape(N, 2, 2, H, W, Cout)
    out = out.transpose(0, 3, 1, 4, 2, 5).reshape(N, 2 * H, 2 * W, Cout)
    return out


def _conv1x1(x, w, b, out_dtype):
    N, H, W, C = x.shape
    Cout = w.shape[-1]
    A = x.reshape(N, H * W, C)
    w1 = w.reshape(1, C, Cout)
    out = _conv_call([A], [w1], b, taps=[(0, 0, 0, C, 0, 0, 0)], M=H * W,
                     Cout=Cout, out_dtype=out_dtype)
    return out.reshape(N, H, W, Cout)


def _block(x, p, comb, *, up, J1=1, J2=1, Jt=1):
    h = _conv3_pair(x, p, comb, J=J2)
    if up:
        return _transconv(h, p['trans_w'], p['trans_b'], J=Jt)
    return _down4(h, p['trans_w'], p['trans_b'], J=Jt)


def kernel(time_mlp_w, time_mlp_b, label_emb, conv0_w, conv0_b, out_w, out_b, down0_conv1_w, down0_conv1_b, down0_conv2_w, down0_conv2_b, down0_time_w, down0_time_b, down0_trans_w, down0_trans_b, down0_gn1_g, down0_gn1_b, down0_gn2_g, down0_gn2_b, down1_conv1_w, down1_conv1_b, down1_conv2_w, down1_conv2_b, down1_time_w, down1_time_b, down1_trans_w, down1_trans_b, down1_gn1_g, down1_gn1_b, down1_gn2_g, down1_gn2_b, down2_conv1_w, down2_conv1_b, down2_conv2_w, down2_conv2_b, down2_time_w, down2_time_b, down2_trans_w, down2_trans_b, down2_gn1_g, down2_gn1_b, down2_gn2_g, down2_gn2_b, down3_conv1_w, down3_conv1_b, down3_conv2_w, down3_conv2_b, down3_time_w, down3_time_b, down3_trans_w, down3_trans_b, down3_gn1_g, down3_gn1_b, down3_gn2_g, down3_gn2_b, up0_conv1_w, up0_conv1_b, up0_conv2_w, up0_conv2_b, up0_time_w, up0_time_b, up0_trans_b, up0_gn1_g, up0_gn1_b, up0_gn2_g, up0_gn2_b, up0_trans_w_0_0, up0_trans_w_0_1, up0_trans_w_1_0, up0_trans_w_1_1, up1_conv1_w, up1_conv1_b, up1_conv2_w, up1_conv2_b, up1_time_w, up1_time_b, up1_trans_b, up1_gn1_g, up1_gn1_b, up1_gn2_g, up1_gn2_b, up1_trans_w_0_0, up1_trans_w_0_1, up1_trans_w_1_0, up1_trans_w_1_1, up2_conv1_w, up2_conv1_b, up2_conv2_w, up2_conv2_b, up2_time_w, up2_time_b, up2_trans_b, up2_gn1_g, up2_gn1_b, up2_gn2_g, up2_gn2_b, up2_trans_w_0_0, up2_trans_w_0_1, up2_trans_w_1_0, up2_trans_w_1_1, up3_conv1_w, up3_conv1_b, up3_conv2_w, up3_conv2_b, up3_time_w, up3_time_b, up3_trans_b, up3_gn1_g, up3_gn1_b, up3_gn2_g, up3_gn2_b, up3_trans_w_0_0, up3_trans_w_0_1, up3_trans_w_1_0, up3_trans_w_1_1, x, timestep, y):
    downs = [
        dict(conv1_w=down0_conv1_w, conv1_b=down0_conv1_b, conv2_w=down0_conv2_w,
             conv2_b=down0_conv2_b, time_w=down0_time_w, time_b=down0_time_b,
             trans_w=down0_trans_w, trans_b=down0_trans_b, gn1_g=down0_gn1_g,
             gn1_b=down0_gn1_b, gn2_g=down0_gn2_g, gn2_b=down0_gn2_b),
        dict(conv1_w=down1_conv1_w, conv1_b=down1_conv1_b, conv2_w=down1_conv2_w,
             conv2_b=down1_conv2_b, time_w=down1_time_w, time_b=down1_time_b,
             trans_w=down1_trans_w, trans_b=down1_trans_b, gn1_g=down1_gn1_g,
             gn1_b=down1_gn1_b, gn2_g=down1_gn2_g, gn2_b=down1_gn2_b),
        dict(conv1_w=down2_conv1_w, conv1_b=down2_conv1_b, conv2_w=down2_conv2_w,
             conv2_b=down2_conv2_b, time_w=down2_time_w, time_b=down2_time_b,
             trans_w=down2_trans_w, trans_b=down2_trans_b, gn1_g=down2_gn1_g,
             gn1_b=down2_gn1_b, gn2_g=down2_gn2_g, gn2_b=down2_gn2_b),
        dict(conv1_w=down3_conv1_w, conv1_b=down3_conv1_b, conv2_w=down3_conv2_w,
             conv2_b=down3_conv2_b, time_w=down3_time_w, time_b=down3_time_b,
             trans_w=down3_trans_w, trans_b=down3_trans_b, gn1_g=down3_gn1_g,
             gn1_b=down3_gn1_b, gn2_g=down3_gn2_g, gn2_b=down3_gn2_b),
    ]
    ups = [
        dict(conv1_w=up0_conv1_w, conv1_b=up0_conv1_b, conv2_w=up0_conv2_w,
             conv2_b=up0_conv2_b, time_w=up0_time_w, time_b=up0_time_b,
             trans_w=[up0_trans_w_0_0, up0_trans_w_0_1, up0_trans_w_1_0,
                      up0_trans_w_1_1], trans_b=up0_trans_b, gn1_g=up0_gn1_g,
             gn1_b=up0_gn1_b, gn2_g=up0_gn2_g, gn2_b=up0_gn2_b),
        dict(conv1_w=up1_conv1_w, conv1_b=up1_conv1_b, conv2_w=up1_conv2_w,
             conv2_b=up1_conv2_b, time_w=up1_time_w, time_b=up1_time_b,
             trans_w=[up1_trans_w_0_0, up1_trans_w_0_1, up1_trans_w_1_0,
                      up1_trans_w_1_1], trans_b=up1_trans_b, gn1_g=up1_gn1_g,
             gn1_b=up1_gn1_b, gn2_g=up1_gn2_g, gn2_b=up1_gn2_b),
        dict(conv1_w=up2_conv1_w, conv1_b=up2_conv1_b, conv2_w=up2_conv2_w,
             conv2_b=up2_conv2_b, time_w=up2_time_w, time_b=up2_time_b,
             trans_w=[up2_trans_w_0_0, up2_trans_w_0_1, up2_trans_w_1_0,
                      up2_trans_w_1_1], trans_b=up2_trans_b, gn1_g=up2_gn1_g,
             gn1_b=up2_gn1_b, gn2_g=up2_gn2_g, gn2_b=up2_gn2_b),
        dict(conv1_w=up3_conv1_w, conv1_b=up3_conv1_b, conv2_w=up3_conv2_w,
             conv2_b=up3_conv2_b, time_w=up3_time_w, time_b=up3_time_b,
             trans_w=[up3_trans_w_0_0, up3_trans_w_0_1, up3_trans_w_1_0,
                      up3_trans_w_1_1], trans_b=up3_trans_b, gn1_g=up3_gn1_g,
             gn1_b=up3_gn1_b, gn2_g=up3_gn2_g, gn2_b=up3_gn2_b),
    ]

    xh = jnp.transpose(x, (0, 2, 3, 1)).astype(jnp.bfloat16)

    half = 16
    freqs = jnp.exp(jnp.arange(half, dtype=jnp.float32)
                    * -(math.log(10000.0) / (half - 1)))
    targs = timestep.astype(jnp.float32)[:, None] * freqs[None, :]
    t_emb = jnp.concatenate([jnp.sin(targs), jnp.cos(targs)], axis=-1)
    te = jnp.dot(t_emb.astype(jnp.bfloat16), time_mlp_w,
                 preferred_element_type=jnp.float32) \
        + time_mlp_b.astype(jnp.float32)[None, :]
    te = jnp.maximum(te, 0.0)
    comb = (te + label_emb[y]).astype(jnp.bfloat16)

    h = _conv3(xh, conv0_w, conv0_b, relu=False)

    down_J = [(1, 1, 1), (1, 1, 1), (1, 1, 1), (1, 2, 2)]
    up_J = [(2, 1, 1), (1, 1, 1), (1, 1, 1), (1, 1, 1)]

    residuals = []
    for p, (j1, j2, jt) in zip(downs, down_J):
        h = _block(h, p, comb, up=False, J1=j1, J2=j2, Jt=jt)
        residuals.append(h)
    for p, (j1, j2, jt) in zip(ups, up_J):
        r = residuals.pop()
        h = _block(jnp.concatenate([h, r], axis=-1), p, comb, up=True,
                   J1=j1, J2=j2, Jt=jt)

    out = _conv1x1(h, out_w, out_b, jnp.float32)
    return jnp.transpose(out, (0, 3, 1, 2))


# speedup vs baseline: 1.1163x; 1.0965x over previous
import functools
import math

import jax
import jax.numpy as jnp
from jax.experimental import pallas as pl
from jax.experimental.pallas import tpu as pltpu

_VMEM_LIMIT = 56 * 1024 * 1024
_EPS = 1e-5


def _shift3_vmem(xb, W):
    M, C = xb.shape
    zc = jnp.zeros((1, C), xb.dtype)
    xz = jnp.concatenate([zc, xb, zc], axis=0)
    jm = jax.lax.broadcasted_iota(jnp.int32, (M, 1), 0) % W
    s0 = jnp.where(jm != 0, xz[0:M], jnp.zeros_like(xb))
    s2 = jnp.where(jm != W - 1, xz[2:M + 2], jnp.zeros_like(xb))
    h3 = jnp.concatenate([s0, xb, s2], axis=1)
    zr = jnp.zeros((W, 3 * C), xb.dtype)
    return jnp.concatenate([zr, h3, zr], axis=0)


def _conv_body(*refs, n_a, n_w, taps, M, relu, gn, temb, n_par, cg,
               build_w=0):
    idx = 0
    a_refs = refs[idx:idx + n_a]; idx += n_a
    w_refs = refs[idx:idx + n_w]; idx += n_w
    b_ref = refs[idx]; idx += 1
    if temb:
        comb_ref, tw_ref, tb_ref = refs[idx:idx + 3]; idx += 3
    if gn:
        g_ref, be_ref = refs[idx:idx + 2]; idx += 2
    o_ref = refs[idx]
    tn = o_ref.shape[-1]

    built = _shift3_vmem(a_refs[0][0], build_w) if build_w else None
    accs = [None] * n_par
    for (ai, ro, co, cl, wi, ws, par) in taps:
        if build_w and ai == 0:
            a = built[ro:ro + M, co:co + cl]
        else:
            a = a_refs[ai][0, ro:ro + M, co:co + cl]
        w = w_refs[wi][ws]
        d = jnp.dot(a, w, preferred_element_type=jnp.float32)
        accs[par] = d if accs[par] is None else accs[par] + d

    bias = b_ref[...]
    for par in range(n_par):
        z = accs[par] + bias
        if relu:
            z = jnp.maximum(z, 0.0)
        if gn:
            zb = z.astype(jnp.bfloat16).astype(jnp.float32)
            s1 = jnp.sum(zb, axis=0, keepdims=True)
            s2 = jnp.sum(zb * zb, axis=0, keepdims=True)
            grp = jax.lax.broadcasted_iota(jnp.int32, (1, tn), 1) // cg
            inv_n = 1.0 / float(M * cg)
            mean = jnp.zeros((1, tn), jnp.float32)
            ex2 = jnp.zeros((1, tn), jnp.float32)
            for g in range(tn // cg):
                m = grp == g
                gs1 = jnp.sum(jnp.where(m, s1, 0.0), axis=-1, keepdims=True)
                gs2 = jnp.sum(jnp.where(m, s2, 0.0), axis=-1, keepdims=True)
                mean = jnp.where(m, gs1 * inv_n, mean)
                ex2 = jnp.where(m, gs2 * inv_n, ex2)
            var = jnp.maximum(ex2 - mean * mean, 0.0)
            inv_std = jax.lax.rsqrt(var + _EPS)
            z = (zb - mean) * inv_std * g_ref[...] + be_ref[...]
            if temb:
                t = jnp.dot(comb_ref[0], tw_ref[...],
                            preferred_element_type=jnp.float32) + tb_ref[...]
                t = jnp.maximum(t, 0.0).astype(jnp.bfloat16).astype(jnp.float32)
                z = z + t
        o_ref[0, par] = z.astype(o_ref.dtype)


def _conv_call(a_list, w_list, bias, *, taps, M, Cout, relu=False, gn=None,
               temb=None, n_par=1, J=1, out_dtype=jnp.bfloat16, build_w=0):
    N = a_list[0].shape[0]
    tn = Cout // J
    bl = J == 1
    ix = (lambda f: (lambda i, j: f(j, i))) if bl else (lambda f: f)
    grid = (N, J) if bl else (J, N)
    in_specs = []
    args = []
    for a in a_list:
        R, K = a.shape[1], a.shape[2]
        in_specs.append(pl.BlockSpec((1, R, K), ix(lambda j, i: (i, 0, 0))))
        args.append(a)
    for w in w_list:
        T, K = w.shape[0], w.shape[1]
        in_specs.append(pl.BlockSpec((T, K, tn), ix(lambda j, i: (0, 0, j))))
        args.append(w)
    in_specs.append(pl.BlockSpec((1, tn), ix(lambda j, i: (0, j))))
    args.append(bias.astype(jnp.float32).reshape(1, Cout))
    if temb is not None:
        comb, tw, tb = temb
        in_specs += [pl.BlockSpec((1, 1, 32), ix(lambda j, i: (i, 0, 0))),
                     pl.BlockSpec((32, tn), ix(lambda j, i: (0, j))),
                     pl.BlockSpec((1, tn), ix(lambda j, i: (0, j)))]
        args += [comb.reshape(N, 1, 32), tw,
                 tb.astype(jnp.float32).reshape(1, Cout)]
    cg = 0
    if gn is not None:
        g, b = gn
        cg = Cout // 8
        in_specs += [pl.BlockSpec((1, tn), ix(lambda j, i: (0, j))),
                     pl.BlockSpec((1, tn), ix(lambda j, i: (0, j)))]
        args += [g.astype(jnp.float32).reshape(1, Cout),
                 b.astype(jnp.float32).reshape(1, Cout)]
    return pl.pallas_call(
        functools.partial(_conv_body, n_a=len(a_list), n_w=len(w_list),
                          taps=taps, M=M, relu=relu, gn=gn is not None,
                          temb=temb is not None, n_par=n_par, cg=cg,
                          build_w=build_w),
        out_shape=jax.ShapeDtypeStruct((N, n_par, M, Cout), out_dtype),
        grid=grid,
        in_specs=in_specs,
        out_specs=pl.BlockSpec((1, n_par, M, tn), ix(lambda j, i: (i, 0, 0, j))),
        compiler_params=pltpu.CompilerParams(
            dimension_semantics=("parallel", "parallel"),
            vmem_limit_bytes=_VMEM_LIMIT),
    )(*args)


def _gn_epilogue(z, g, be, tn, cg):
    zb = z.astype(jnp.bfloat16).astype(jnp.float32)
    M = zb.shape[0]
    s1 = jnp.sum(zb, axis=0, keepdims=True)
    s2 = jnp.sum(zb * zb, axis=0, keepdims=True)
    grp = jax.lax.broadcasted_iota(jnp.int32, (1, tn), 1) // cg
    inv_n = 1.0 / float(M * cg)
    mean = jnp.zeros((1, tn), jnp.float32)
    ex2 = jnp.zeros((1, tn), jnp.float32)
    for g_i in range(tn // cg):
        m = grp == g_i
        gs1 = jnp.sum(jnp.where(m, s1, 0.0), axis=-1, keepdims=True)
        gs2 = jnp.sum(jnp.where(m, s2, 0.0), axis=-1, keepdims=True)
        mean = jnp.where(m, gs1 * inv_n, mean)
        ex2 = jnp.where(m, gs2 * inv_n, ex2)
    var = jnp.maximum(ex2 - mean * mean, 0.0)
    inv_std = jax.lax.rsqrt(var + _EPS)
    return (zb - mean) * inv_std * g + be


def _dbl_body(a_ref, w1_ref, w2_ref, b1_ref, comb_ref, tw_ref, tb_ref,
              g1_ref, be1_ref, b2_ref, g2_ref, be2_ref, o_ref, *, M, W, C1,
              cg2, pack1, pack2):
    x3z = _shift3_vmem(a_ref[0], W)
    if pack1:
        P = jnp.dot(x3z, w1_ref[...], preferred_element_type=jnp.float32)
        acc = (P[0:M, 0:C1] + P[W:W + M, C1:2 * C1]
               + P[2 * W:2 * W + M, 2 * C1:3 * C1])
    else:
        acc = None
        for dh in range(3):
            d = jnp.dot(x3z[dh * W:dh * W + M, :], w1_ref[dh],
                        preferred_element_type=jnp.float32)
            acc = d if acc is None else acc + d
    z = jnp.maximum(acc + b1_ref[...], 0.0)
    h = _gn_epilogue(z, g1_ref[...], be1_ref[...], C1, C1 // 8)
    t = jnp.dot(comb_ref[0], tw_ref[...],
                preferred_element_type=jnp.float32) + tb_ref[...]
    t = jnp.maximum(t, 0.0).astype(jnp.bfloat16).astype(jnp.float32)
    h1b = (h + t).astype(jnp.bfloat16)

    h3z = _shift3_vmem(h1b, W)

    tn = o_ref.shape[-1]
    if pack2:
        P2 = jnp.dot(h3z, w2_ref[...], preferred_element_type=jnp.float32)
        acc2 = (P2[0:M, 0:tn] + P2[W:W + M, tn:2 * tn]
                + P2[2 * W:2 * W + M, 2 * tn:3 * tn])
    else:
        acc2 = None
        for dh in range(3):
            d = jnp.dot(h3z[dh * W:dh * W + M, :], w2_ref[dh],
                        preferred_element_type=jnp.float32)
            acc2 = d if acc2 is None else acc2 + d
    z2 = jnp.maximum(acc2 + b2_ref[...], 0.0)
    out = _gn_epilogue(z2, g2_ref[...], be2_ref[...], tn, cg2)
    o_ref[0, 0] = out.astype(o_ref.dtype)


def _pad_shift3(x):
    N, H, W, C = x.shape
    xp = jnp.pad(x, ((0, 0), (1, 1), (1, 1), (0, 0)))
    xs = jnp.concatenate([xp[:, :, d:d + W, :] for d in range(3)], axis=-1)
    return xs.reshape(N, (H + 2) * W, 3 * C)


def _conv3(x, w, b, *, relu, gn=None, temb=None, J=1):
    N, H, W, C = x.shape
    Cout = w.shape[-1]
    A = _pad_shift3(x)
    w3 = w.reshape(3, 3 * C, Cout)
    taps = [(0, d * W, 0, 3 * C, 0, d, 0) for d in range(3)]
    out = _conv_call([A], [w3], b, taps=taps, M=H * W, Cout=Cout, relu=relu,
                     gn=gn, temb=temb, J=J)
    return out.reshape(N, H, W, Cout)


def _conv3_pair(x, p, comb, *, J=1):
    N, H, W, Cin = x.shape
    C1 = p['conv1_w'].shape[-1]
    C2 = p['conv2_w'].shape[-1]
    M = H * W
    tn = C2 // J
    bl = J == 1
    ix = (lambda f: (lambda i, j: f(j, i))) if bl else (lambda f: f)
    grid = (N, J) if bl else (J, N)
    A = x.reshape(N, M, Cin)
    K1 = 3 * Cin
    pack1 = C1 <= 128
    pack2 = C2 <= 128 and J == 1
    w1 = p['conv1_w'].reshape(3, K1, C1)
    w2 = p['conv2_w'].reshape(3, 3 * C1, C2)
    if pack1:
        w1 = jnp.transpose(w1, (1, 0, 2)).reshape(K1, 3 * C1)
        w1_spec = pl.BlockSpec((K1, 3 * C1), ix(lambda j, i: (0, 0)))
    else:
        w1_spec = pl.BlockSpec((3, K1, C1), ix(lambda j, i: (0, 0, 0)))
    if pack2:
        w2 = jnp.transpose(w2, (1, 0, 2)).reshape(3 * C1, 3 * C2)
        w2_spec = pl.BlockSpec((3 * C1, 3 * C2), ix(lambda j, i: (0, 0)))
    else:
        w2_spec = pl.BlockSpec((3, 3 * C1, tn), ix(lambda j, i: (0, 0, j)))
    f32 = lambda a: a.astype(jnp.float32)
    args = [A, w1, w2,
            f32(p['conv1_b']).reshape(1, C1),
            comb.reshape(N, 1, 32), p['time_w'],
            f32(p['time_b']).reshape(1, C1),
            f32(p['gn1_g']).reshape(1, C1), f32(p['gn1_b']).reshape(1, C1),
            f32(p['conv2_b']).reshape(1, C2),
            f32(p['gn2_g']).reshape(1, C2), f32(p['gn2_b']).reshape(1, C2)]
    in_specs = [
        pl.BlockSpec((1, M, Cin), ix(lambda j, i: (i, 0, 0))),
        w1_spec,
        w2_spec,
        pl.BlockSpec((1, C1), ix(lambda j, i: (0, 0))),
        pl.BlockSpec((1, 1, 32), ix(lambda j, i: (i, 0, 0))),
        pl.BlockSpec((32, C1), ix(lambda j, i: (0, 0))),
        pl.BlockSpec((1, C1), ix(lambda j, i: (0, 0))),
        pl.BlockSpec((1, C1), ix(lambda j, i: (0, 0))),
        pl.BlockSpec((1, C1), ix(lambda j, i: (0, 0))),
        pl.BlockSpec((1, tn), ix(lambda j, i: (0, j))),
        pl.BlockSpec((1, tn), ix(lambda j, i: (0, j))),
        pl.BlockSpec((1, tn), ix(lambda j, i: (0, j))),
    ]
    out = pl.pallas_call(
        functools.partial(_dbl_body, M=M, W=W, C1=C1, cg2=C2 // 8,
                          pack1=pack1, pack2=pack2),
        out_shape=jax.ShapeDtypeStruct((N, 1, M, C2), jnp.bfloat16),
        grid=grid,
        in_specs=in_specs,
        out_specs=pl.BlockSpec((1, 1, M, tn), ix(lambda j, i: (i, 0, 0, j))),
        compiler_params=pltpu.CompilerParams(
            dimension_semantics=("parallel", "parallel"),
            vmem_limit_bytes=_VMEM_LIMIT),
    )(*args)
    return out.reshape(N, H, W, C2)


def _down4(x, w, b, *, J=1):
    N, H, W, C = x.shape
    Ho, Wo = H // 2, W // 2
    Cout = w.shape[-1]
    xp = jnp.pad(x, ((0, 0), (1, 2), (1, 2), (0, 0)))
    xs = jnp.concatenate([xp[:, :, d::2, :][:, :, :Wo, :] for d in range(4)],
                         axis=-1)
    A0 = xs[:, 0::2].reshape(N, -1, 4 * C)
    A1 = xs[:, 1::2].reshape(N, -1, 4 * C)
    w4 = w.reshape(4, 4 * C, Cout)
    taps = [(d % 2, (d // 2) * Wo, 0, 4 * C, 0, d, 0) for d in range(4)]
    out = _conv_call([A0, A1], [w4], b, taps=taps, M=Ho * Wo, Cout=Cout, J=J)
    return out.reshape(N, Ho, Wo, Cout)


def _transconv(x, wlist4, b, *, J=1):
    N, H, W, C = x.shape
    Cout = wlist4[0].shape[-1]
    A = x.reshape(N, H * W, C)
    ws = [w.reshape(2, 2 * C, Cout) for w in wlist4]
    taps = []
    for r in range(2):
        for c in range(2):
            p = r * 2 + c
            for a in range(2):
                taps.append((0, (r + a) * W, c * C, 2 * C, p, a, p))
    out = _conv_call([A], ws, b, taps=taps, M=H * W, Cout=Cout, n_par=4, J=J,
                     build_w=W)
    out = out.reshape(N, 2, 2, H, W, Cout)
    out = out.transpose(0, 3, 1, 4, 2, 5).reshape(N, 2 * H, 2 * W, Cout)
    return out


def _conv1x1(x, w, b, out_dtype):
    N, H, W, C = x.shape
    Cout = w.shape[-1]
    A = x.reshape(N, H * W, C)
    w1 = w.reshape(1, C, Cout)
    out = _conv_call([A], [w1], b, taps=[(0, 0, 0, C, 0, 0, 0)], M=H * W,
                     Cout=Cout, out_dtype=out_dtype)
    return out.reshape(N, H, W, Cout)


def _block(x, p, comb, *, up, J1=1, J2=1, Jt=1):
    h = _conv3_pair(x, p, comb, J=J2)
    if up:
        return _transconv(h, p['trans_w'], p['trans_b'], J=Jt)
    return _down4(h, p['trans_w'], p['trans_b'], J=Jt)


def kernel(time_mlp_w, time_mlp_b, label_emb, conv0_w, conv0_b, out_w, out_b, down0_conv1_w, down0_conv1_b, down0_conv2_w, down0_conv2_b, down0_time_w, down0_time_b, down0_trans_w, down0_trans_b, down0_gn1_g, down0_gn1_b, down0_gn2_g, down0_gn2_b, down1_conv1_w, down1_conv1_b, down1_conv2_w, down1_conv2_b, down1_time_w, down1_time_b, down1_trans_w, down1_trans_b, down1_gn1_g, down1_gn1_b, down1_gn2_g, down1_gn2_b, down2_conv1_w, down2_conv1_b, down2_conv2_w, down2_conv2_b, down2_time_w, down2_time_b, down2_trans_w, down2_trans_b, down2_gn1_g, down2_gn1_b, down2_gn2_g, down2_gn2_b, down3_conv1_w, down3_conv1_b, down3_conv2_w, down3_conv2_b, down3_time_w, down3_time_b, down3_trans_w, down3_trans_b, down3_gn1_g, down3_gn1_b, down3_gn2_g, down3_gn2_b, up0_conv1_w, up0_conv1_b, up0_conv2_w, up0_conv2_b, up0_time_w, up0_time_b, up0_trans_b, up0_gn1_g, up0_gn1_b, up0_gn2_g, up0_gn2_b, up0_trans_w_0_0, up0_trans_w_0_1, up0_trans_w_1_0, up0_trans_w_1_1, up1_conv1_w, up1_conv1_b, up1_conv2_w, up1_conv2_b, up1_time_w, up1_time_b, up1_trans_b, up1_gn1_g, up1_gn1_b, up1_gn2_g, up1_gn2_b, up1_trans_w_0_0, up1_trans_w_0_1, up1_trans_w_1_0, up1_trans_w_1_1, up2_conv1_w, up2_conv1_b, up2_conv2_w, up2_conv2_b, up2_time_w, up2_time_b, up2_trans_b, up2_gn1_g, up2_gn1_b, up2_gn2_g, up2_gn2_b, up2_trans_w_0_0, up2_trans_w_0_1, up2_trans_w_1_0, up2_trans_w_1_1, up3_conv1_w, up3_conv1_b, up3_conv2_w, up3_conv2_b, up3_time_w, up3_time_b, up3_trans_b, up3_gn1_g, up3_gn1_b, up3_gn2_g, up3_gn2_b, up3_trans_w_0_0, up3_trans_w_0_1, up3_trans_w_1_0, up3_trans_w_1_1, x, timestep, y):
    downs = [
        dict(conv1_w=down0_conv1_w, conv1_b=down0_conv1_b, conv2_w=down0_conv2_w,
             conv2_b=down0_conv2_b, time_w=down0_time_w, time_b=down0_time_b,
             trans_w=down0_trans_w, trans_b=down0_trans_b, gn1_g=down0_gn1_g,
             gn1_b=down0_gn1_b, gn2_g=down0_gn2_g, gn2_b=down0_gn2_b),
        dict(conv1_w=down1_conv1_w, conv1_b=down1_conv1_b, conv2_w=down1_conv2_w,
             conv2_b=down1_conv2_b, time_w=down1_time_w, time_b=down1_time_b,
             trans_w=down1_trans_w, trans_b=down1_trans_b, gn1_g=down1_gn1_g,
             gn1_b=down1_gn1_b, gn2_g=down1_gn2_g, gn2_b=down1_gn2_b),
        dict(conv1_w=down2_conv1_w, conv1_b=down2_conv1_b, conv2_w=down2_conv2_w,
             conv2_b=down2_conv2_b, time_w=down2_time_w, time_b=down2_time_b,
             trans_w=down2_trans_w, trans_b=down2_trans_b, gn1_g=down2_gn1_g,
             gn1_b=down2_gn1_b, gn2_g=down2_gn2_g, gn2_b=down2_gn2_b),
        dict(conv1_w=down3_conv1_w, conv1_b=down3_conv1_b, conv2_w=down3_conv2_w,
             conv2_b=down3_conv2_b, time_w=down3_time_w, time_b=down3_time_b,
             trans_w=down3_trans_w, trans_b=down3_trans_b, gn1_g=down3_gn1_g,
             gn1_b=down3_gn1_b, gn2_g=down3_gn2_g, gn2_b=down3_gn2_b),
    ]
    ups = [
        dict(conv1_w=up0_conv1_w, conv1_b=up0_conv1_b, conv2_w=up0_conv2_w,
             conv2_b=up0_conv2_b, time_w=up0_time_w, time_b=up0_time_b,
             trans_w=[up0_trans_w_0_0, up0_trans_w_0_1, up0_trans_w_1_0,
                      up0_trans_w_1_1], trans_b=up0_trans_b, gn1_g=up0_gn1_g,
             gn1_b=up0_gn1_b, gn2_g=up0_gn2_g, gn2_b=up0_gn2_b),
        dict(conv1_w=up1_conv1_w, conv1_b=up1_conv1_b, conv2_w=up1_conv2_w,
             conv2_b=up1_conv2_b, time_w=up1_time_w, time_b=up1_time_b,
             trans_w=[up1_trans_w_0_0, up1_trans_w_0_1, up1_trans_w_1_0,
                      up1_trans_w_1_1], trans_b=up1_trans_b, gn1_g=up1_gn1_g,
             gn1_b=up1_gn1_b, gn2_g=up1_gn2_g, gn2_b=up1_gn2_b),
        dict(conv1_w=up2_conv1_w, conv1_b=up2_conv1_b, conv2_w=up2_conv2_w,
             conv2_b=up2_conv2_b, time_w=up2_time_w, time_b=up2_time_b,
             trans_w=[up2_trans_w_0_0, up2_trans_w_0_1, up2_trans_w_1_0,
                      up2_trans_w_1_1], trans_b=up2_trans_b, gn1_g=up2_gn1_g,
             gn1_b=up2_gn1_b, gn2_g=up2_gn2_g, gn2_b=up2_gn2_b),
        dict(conv1_w=up3_conv1_w, conv1_b=up3_conv1_b, conv2_w=up3_conv2_w,
             conv2_b=up3_conv2_b, time_w=up3_time_w, time_b=up3_time_b,
             trans_w=[up3_trans_w_0_0, up3_trans_w_0_1, up3_trans_w_1_0,
                      up3_trans_w_1_1], trans_b=up3_trans_b, gn1_g=up3_gn1_g,
             gn1_b=up3_gn1_b, gn2_g=up3_gn2_g, gn2_b=up3_gn2_b),
    ]

    xh = jnp.transpose(x, (0, 2, 3, 1)).astype(jnp.bfloat16)

    half = 16
    freqs = jnp.exp(jnp.arange(half, dtype=jnp.float32)
                    * -(math.log(10000.0) / (half - 1)))
    targs = timestep.astype(jnp.float32)[:, None] * freqs[None, :]
    t_emb = jnp.concatenate([jnp.sin(targs), jnp.cos(targs)], axis=-1)
    te = jnp.dot(t_emb.astype(jnp.bfloat16), time_mlp_w,
                 preferred_element_type=jnp.float32) \
        + time_mlp_b.astype(jnp.float32)[None, :]
    te = jnp.maximum(te, 0.0)
    comb = (te + label_emb[y]).astype(jnp.bfloat16)

    h = _conv3(xh, conv0_w, conv0_b, relu=False)

    down_J = [(1, 1, 1), (1, 1, 1), (1, 1, 1), (1, 2, 2)]
    up_J = [(2, 1, 1), (1, 1, 1), (1, 1, 1), (1, 1, 1)]

    residuals = []
    for p, (j1, j2, jt) in zip(downs, down_J):
        h = _block(h, p, comb, up=False, J1=j1, J2=j2, Jt=jt)
        residuals.append(h)
    for p, (j1, j2, jt) in zip(ups, up_J):
        r = residuals.pop()
        h = _block(jnp.concatenate([h, r], axis=-1), p, comb, up=True,
                   J1=j1, J2=j2, Jt=jt)

    out = _conv1x1(h, out_w, out_b, jnp.float32)
    return jnp.transpose(out, (0, 3, 1, 2))
